# initial kernel scaffold (unmeasured)
import jax
import jax.numpy as jnp
from jax import lax
from jax.experimental import pallas as pl
from jax.experimental.pallas import tpu as pltpu


def kernel(A, B):
    m, k = A.shape
    _, n = B.shape

    def body(a_ref, b_ref, out_ref, send_buf, recv_buf, send_sem, recv_sem):
        my_x = lax.axis_index("x")
        my_y = lax.axis_index("y")
        peer = (my_x, 1 - my_y)

        barrier_sem = pltpu.get_barrier_semaphore()
        pl.semaphore_signal(
            barrier_sem, inc=1, device_id=peer,
            device_id_type=pl.DeviceIdType.MESH,
        )
        pl.semaphore_wait(barrier_sem, 1)

        send_buf[...] = jnp.dot(
            a_ref[...], b_ref[...], preferred_element_type=jnp.float32
        )
        rdma = pltpu.make_async_remote_copy(
            src_ref=send_buf,
            dst_ref=recv_buf,
            send_sem=send_sem,
            recv_sem=recv_sem,
            device_id=peer,
            device_id_type=pl.DeviceIdType.MESH,
        )
        rdma.start()
        rdma.wait()
        out_ref[...] = send_buf[...] + recv_buf[...]

    return pl.pallas_call(
        body,
        out_shape=jax.ShapeDtypeStruct((m, n), jnp.float32),
        in_specs=[
            pl.BlockSpec(memory_space=pltpu.VMEM),
            pl.BlockSpec(memory_space=pltpu.VMEM),
        ],
        out_specs=pl.BlockSpec(memory_space=pltpu.VMEM),
        scratch_shapes=[
            pltpu.VMEM((m, n), jnp.float32),
            pltpu.VMEM((m, n), jnp.float32),
            pltpu.SemaphoreType.DMA,
            pltpu.SemaphoreType.DMA,
        ],
        compiler_params=pltpu.CompilerParams(collective_id=0),
    )(A, B)


# baseline (device time: 218792 ns/iter reference)
import jax
import jax.numpy as jnp
from jax import lax
from jax.experimental import pallas as pl
from jax.experimental.pallas import tpu as pltpu


def kernel(A, B):
    m, k = A.shape
    _, n = B.shape

    def body(a_ref, b_ref, out_ref, recv_buf, send_sem, recv_sem):
        my_x = lax.axis_index("x")
        my_y = lax.axis_index("y")
        peer = (my_x, 1 - my_y)

        barrier_sem = pltpu.get_barrier_semaphore()
        pl.semaphore_signal(
            barrier_sem, inc=1, device_id=peer,
            device_id_type=pl.DeviceIdType.MESH,
        )
        pl.semaphore_wait(barrier_sem, 1)

        out_ref[...] = jnp.dot(
            a_ref[...], b_ref[...], preferred_element_type=jnp.float32
        )
        rdma = pltpu.make_async_remote_copy(
            src_ref=out_ref,
            dst_ref=recv_buf,
            send_sem=send_sem,
            recv_sem=recv_sem,
            device_id=peer,
            device_id_type=pl.DeviceIdType.MESH,
        )
        rdma.start()
        rdma.wait()
        out_ref[...] = out_ref[...] + recv_buf[...]

    return pl.pallas_call(
        body,
        out_shape=jax.ShapeDtypeStruct((m, n), jnp.float32),
        in_specs=[
            pl.BlockSpec(memory_space=pltpu.VMEM),
            pl.BlockSpec(memory_space=pltpu.VMEM),
        ],
        out_specs=pl.BlockSpec(memory_space=pltpu.VMEM),
        scratch_shapes=[
            pltpu.VMEM((m, n), jnp.float32),
            pltpu.SemaphoreType.DMA,
            pltpu.SemaphoreType.DMA,
        ],
        compiler_params=pltpu.CompilerParams(
            collective_id=0,
            vmem_limit_bytes=100 * 1024 * 1024,
        ),
    )(A, B)


# device time: 209265 ns/iter; 1.0455x vs baseline; 1.0455x over previous
import jax
import jax.numpy as jnp
from jax import lax
from jax.experimental import pallas as pl
from jax.experimental.pallas import tpu as pltpu

N_CHUNKS = 8


def kernel(A, B):
    m, k = A.shape
    _, n = B.shape
    mc = m // N_CHUNKS

    def body(a_ref, b_ref, out_ref, recv_buf, send_sems, recv_sems):
        my_x = lax.axis_index("x")
        my_y = lax.axis_index("y")
        peer = (my_x, 1 - my_y)

        barrier_sem = pltpu.get_barrier_semaphore()
        pl.semaphore_signal(
            barrier_sem, inc=1, device_id=peer,
            device_id_type=pl.DeviceIdType.MESH,
        )
        pl.semaphore_wait(barrier_sem, 1)

        rdmas = []
        for i in range(N_CHUNKS):
            sl = pl.ds(i * mc, mc)
            out_ref[sl, :] = jnp.dot(
                a_ref[sl, :], b_ref[...], preferred_element_type=jnp.float32
            )
            rdma = pltpu.make_async_remote_copy(
                src_ref=out_ref.at[sl],
                dst_ref=recv_buf.at[sl],
                send_sem=send_sems.at[i],
                recv_sem=recv_sems.at[i],
                device_id=peer,
                device_id_type=pl.DeviceIdType.MESH,
            )
            rdma.start()
            rdmas.append(rdma)

        for i in range(N_CHUNKS):
            sl = pl.ds(i * mc, mc)
            rdmas[i].wait_send()
            rdmas[i].wait_recv()
            out_ref[sl, :] = out_ref[sl, :] + recv_buf[sl, :]

    return pl.pallas_call(
        body,
        out_shape=jax.ShapeDtypeStruct((m, n), jnp.float32),
        in_specs=[
            pl.BlockSpec(memory_space=pltpu.VMEM),
            pl.BlockSpec(memory_space=pltpu.VMEM),
        ],
        out_specs=pl.BlockSpec(memory_space=pltpu.VMEM),
        scratch_shapes=[
            pltpu.VMEM((m, n), jnp.float32),
            pltpu.SemaphoreType.DMA((N_CHUNKS,)),
            pltpu.SemaphoreType.DMA((N_CHUNKS,)),
        ],
        compiler_params=pltpu.CompilerParams(
            collective_id=0,
            vmem_limit_bytes=100 * 1024 * 1024,
        ),
    )(A, B)


# device time: 119277 ns/iter; 1.8343x vs baseline; 1.7544x over previous
import os

import jax
import jax.numpy as jnp
from jax import lax
from jax.experimental import pallas as pl
from jax.experimental.pallas import tpu as pltpu

N_CHUNKS = int(os.environ.get("N_CHUNKS", "8"))


def kernel(A, B):
    m, k = A.shape
    _, n = B.shape
    mc = m // N_CHUNKS

    def body(a_ref, b_ref, out_ref, send_buf, recv_buf, send_sems, recv_sems):
        my_x = lax.axis_index("x")
        my_y = lax.axis_index("y")
        peer = (my_x, 1 - my_y)

        barrier_sem = pltpu.get_barrier_semaphore()
        pl.semaphore_signal(
            barrier_sem, inc=1, device_id=peer,
            device_id_type=pl.DeviceIdType.MESH,
        )
        pl.semaphore_wait(barrier_sem, 1)

        rdmas = []
        for i in range(N_CHUNKS):
            sl = pl.ds(i * mc, mc)
            out_ref[sl, :] = jnp.dot(
                a_ref[sl, :], b_ref[...], preferred_element_type=jnp.float32
            )
            send_buf[sl, :] = out_ref[sl, :].astype(jnp.bfloat16)
            rdma = pltpu.make_async_remote_copy(
                src_ref=send_buf.at[sl],
                dst_ref=recv_buf.at[sl],
                send_sem=send_sems.at[i],
                recv_sem=recv_sems.at[i],
                device_id=peer,
                device_id_type=pl.DeviceIdType.MESH,
            )
            rdma.start()
            rdmas.append(rdma)

        for i in range(N_CHUNKS):
            sl = pl.ds(i * mc, mc)
            rdmas[i].wait_send()
            rdmas[i].wait_recv()
            out_ref[sl, :] = out_ref[sl, :] + recv_buf[sl, :].astype(
                jnp.float32
            )

    return pl.pallas_call(
        body,
        out_shape=jax.ShapeDtypeStruct((m, n), jnp.float32),
        in_specs=[
            pl.BlockSpec(memory_space=pltpu.VMEM),
            pl.BlockSpec(memory_space=pltpu.VMEM),
        ],
        out_specs=pl.BlockSpec(memory_space=pltpu.VMEM),
        scratch_shapes=[
            pltpu.VMEM((m, n), jnp.bfloat16),
            pltpu.VMEM((m, n), jnp.bfloat16),
            pltpu.SemaphoreType.DMA((N_CHUNKS,)),
            pltpu.SemaphoreType.DMA((N_CHUNKS,)),
        ],
        compiler_params=pltpu.CompilerParams(
            collective_id=0,
            vmem_limit_bytes=100 * 1024 * 1024,
        ),
    )(A, B)


# device time: 102979 ns/iter; 2.1246x vs baseline; 1.1583x over previous
import os

import jax
import jax.numpy as jnp
from jax import lax
from jax.experimental import pallas as pl
from jax.experimental.pallas import tpu as pltpu

N_CHUNKS = int(os.environ.get("N_CHUNKS", "4"))


def kernel(A, B):
    m, k = A.shape
    _, n = B.shape
    hn = n // 2
    mc = m // N_CHUNKS

    def body(
        a_ref, b_ref, out_ref,
        a_send, b_send, a_recv, b_direct, b_fwd,
        a_send_sems, a_recv_sems,
        b_send_sem, b_recv_sem, fwd_send_sem, fwd_recv_sem,
    ):
        my_x = lax.axis_index("x")
        my_y = lax.axis_index("y")
        row_peer = (my_x, 1 - my_y)
        col_peer = (1 - my_x, my_y)

        barrier_sem = pltpu.get_barrier_semaphore()
        for nbr in [row_peer, col_peer]:
            pl.semaphore_signal(
                barrier_sem, inc=1, device_id=nbr,
                device_id_type=pl.DeviceIdType.MESH,
            )
        pl.semaphore_wait(barrier_sem, 2)

        b_send[...] = b_ref[:, pl.ds(my_x * hn, hn)].astype(jnp.bfloat16)
        rdma_b = pltpu.make_async_remote_copy(
            src_ref=b_send, dst_ref=b_direct,
            send_sem=b_send_sem, recv_sem=b_recv_sem,
            device_id=row_peer, device_id_type=pl.DeviceIdType.MESH,
        )
        rdma_b.start()

        rdmas_a = []
        for i in range(N_CHUNKS):
            sl = pl.ds(i * mc, mc)
            a_send[sl, :] = a_ref[sl, :].astype(jnp.bfloat16)
            rd = pltpu.make_async_remote_copy(
                src_ref=a_send.at[sl],
                dst_ref=a_recv.at[sl],
                send_sem=a_send_sems.at[i],
                recv_sem=a_recv_sems.at[i],
                device_id=row_peer, device_id_type=pl.DeviceIdType.MESH,
            )
            rd.start()
            rdmas_a.append(rd)

        out_ref[...] = jnp.dot(
            a_ref[...], b_ref[...], preferred_element_type=jnp.float32
        )

        rdma_b.wait_recv()
        rdma_fwd = pltpu.make_async_remote_copy(
            src_ref=b_direct, dst_ref=b_fwd,
            send_sem=fwd_send_sem, recv_sem=fwd_recv_sem,
            device_id=col_peer, device_id_type=pl.DeviceIdType.MESH,
        )
        rdma_fwd.start()

        csl_d = pl.ds(my_x * hn, hn)
        for i in range(N_CHUNKS):
            sl = pl.ds(i * mc, mc)
            rdmas_a[i].wait_recv()
            out_ref[sl, csl_d] = out_ref[sl, csl_d] + jnp.dot(
                a_recv[sl, :], b_direct[...],
                preferred_element_type=jnp.float32,
            )

        rdma_fwd.wait_recv()
        csl_f = pl.ds((1 - my_x) * hn, hn)
        for i in range(N_CHUNKS):
            sl = pl.ds(i * mc, mc)
            out_ref[sl, csl_f] = out_ref[sl, csl_f] + jnp.dot(
                a_recv[sl, :], b_fwd[...],
                preferred_element_type=jnp.float32,
            )

        rdma_b.wait_send()
        rdma_fwd.wait_send()
        for i in range(N_CHUNKS):
            rdmas_a[i].wait_send()

    return pl.pallas_call(
        body,
        out_shape=jax.ShapeDtypeStruct((m, n), jnp.float32),
        in_specs=[
            pl.BlockSpec(memory_space=pltpu.VMEM),
            pl.BlockSpec(memory_space=pltpu.VMEM),
        ],
        out_specs=pl.BlockSpec(memory_space=pltpu.VMEM),
        scratch_shapes=[
            pltpu.VMEM((m, k), jnp.bfloat16),
            pltpu.VMEM((k, hn), jnp.bfloat16),
            pltpu.VMEM((m, k), jnp.bfloat16),
            pltpu.VMEM((k, hn), jnp.bfloat16),
            pltpu.VMEM((k, hn), jnp.bfloat16),
            pltpu.SemaphoreType.DMA((N_CHUNKS,)),
            pltpu.SemaphoreType.DMA((N_CHUNKS,)),
            pltpu.SemaphoreType.DMA,
            pltpu.SemaphoreType.DMA,
            pltpu.SemaphoreType.DMA,
            pltpu.SemaphoreType.DMA,
        ],
        compiler_params=pltpu.CompilerParams(
            collective_id=0,
            vmem_limit_bytes=100 * 1024 * 1024,
        ),
    )(A, B)


# device time: 99150 ns/iter; 2.2067x vs baseline; 1.0386x over previous
import os

import jax
import jax.numpy as jnp
from jax import lax
from jax.experimental import pallas as pl
from jax.experimental.pallas import tpu as pltpu

N_CHUNKS = int(os.environ.get("N_CHUNKS", "4"))


def kernel(A, B):
    m, k = A.shape
    _, n = B.shape
    hn = n // 2
    mc = m // N_CHUNKS

    def body(
        a_ref, b_ref, out_ref,
        a_send, b_send, a_recv, b_direct, b_fwd,
        a_send_sems, a_recv_sems,
        b_send_sem, b_recv_sem, fwd_send_sem, fwd_recv_sem,
    ):
        my_x = lax.axis_index("x")
        my_y = lax.axis_index("y")
        row_peer = (my_x, 1 - my_y)
        col_peer = (1 - my_x, my_y)

        barrier_sem = pltpu.get_barrier_semaphore()
        for nbr in [row_peer, col_peer]:
            pl.semaphore_signal(
                barrier_sem, inc=1, device_id=nbr,
                device_id_type=pl.DeviceIdType.MESH,
            )
        pl.semaphore_wait(barrier_sem, 2)

        b_send[...] = b_ref[:, pl.ds(my_x * hn, hn)].astype(jnp.bfloat16)
        rdma_b = pltpu.make_async_remote_copy(
            src_ref=b_send, dst_ref=b_direct,
            send_sem=b_send_sem, recv_sem=b_recv_sem,
            device_id=row_peer, device_id_type=pl.DeviceIdType.MESH,
        )
        rdma_b.start()

        rdmas_a = []
        for i in range(N_CHUNKS):
            sl = pl.ds(i * mc, mc)
            a_send[sl, :] = a_ref[sl, :].astype(jnp.bfloat16)
            rd = pltpu.make_async_remote_copy(
                src_ref=a_send.at[sl],
                dst_ref=a_recv.at[sl],
                send_sem=a_send_sems.at[i],
                recv_sem=a_recv_sems.at[i],
                device_id=row_peer, device_id_type=pl.DeviceIdType.MESH,
            )
            rd.start()
            rdmas_a.append(rd)

        out_ref[...] = jnp.dot(
            a_ref[...], b_ref[...], preferred_element_type=jnp.float32
        )

        rdma_b.wait_recv()
        rdma_fwd = pltpu.make_async_remote_copy(
            src_ref=b_direct, dst_ref=b_fwd,
            send_sem=fwd_send_sem, recv_sem=fwd_recv_sem,
            device_id=col_peer, device_id_type=pl.DeviceIdType.MESH,
        )
        rdma_fwd.start()

        csl_d = pl.ds(my_x * hn, hn)
        csl_f = pl.ds((1 - my_x) * hn, hn)
        fwd_at = max(N_CHUNKS - 4, 0)

        def dot2b(j):
            sl = pl.ds(j * mc, mc)
            out_ref[sl, csl_f] = out_ref[sl, csl_f] + jnp.dot(
                a_recv[sl, :], b_fwd[...],
                preferred_element_type=jnp.float32,
            )

        for i in range(N_CHUNKS):
            sl = pl.ds(i * mc, mc)
            rdmas_a[i].wait_recv()
            out_ref[sl, csl_d] = out_ref[sl, csl_d] + jnp.dot(
                a_recv[sl, :], b_direct[...],
                preferred_element_type=jnp.float32,
            )
            if i == fwd_at:
                rdma_fwd.wait_recv()
            if i >= fwd_at:
                dot2b(i - fwd_at)
        for j in range(N_CHUNKS - fwd_at, N_CHUNKS):
            dot2b(j)

        rdma_b.wait_send()
        rdma_fwd.wait_send()
        for i in range(N_CHUNKS):
            rdmas_a[i].wait_send()

    return pl.pallas_call(
        body,
        out_shape=jax.ShapeDtypeStruct((m, n), jnp.float32),
        in_specs=[
            pl.BlockSpec(memory_space=pltpu.VMEM),
            pl.BlockSpec(memory_space=pltpu.VMEM),
        ],
        out_specs=pl.BlockSpec(memory_space=pltpu.VMEM),
        scratch_shapes=[
            pltpu.VMEM((m, k), jnp.bfloat16),
            pltpu.VMEM((k, hn), jnp.bfloat16),
            pltpu.VMEM((m, k), jnp.bfloat16),
            pltpu.VMEM((k, hn), jnp.bfloat16),
            pltpu.VMEM((k, hn), jnp.bfloat16),
            pltpu.SemaphoreType.DMA((N_CHUNKS,)),
            pltpu.SemaphoreType.DMA((N_CHUNKS,)),
            pltpu.SemaphoreType.DMA,
            pltpu.SemaphoreType.DMA,
            pltpu.SemaphoreType.DMA,
            pltpu.SemaphoreType.DMA,
        ],
        compiler_params=pltpu.CompilerParams(
            collective_id=0,
            vmem_limit_bytes=100 * 1024 * 1024,
        ),
    )(A, B)
